# baseline (speedup 1.0000x reference)
import functools

import jax
import jax.numpy as jnp
from jax.experimental import pallas as pl
from jax.experimental.pallas import tpu as pltpu


def _twohot_body(v_ref, b_ref, u1_ref, au1_ref, u2_ref, au2_ref, o_ref):
    v = jnp.maximum(v_ref[...], b_ref[0, 0, 0])
    t1 = jnp.clip(v * u1_ref[...] - au1_ref[...], 0.0, 1.0)
    t2 = jnp.clip(v * u2_ref[...] - au2_ref[...], 0.0, 1.0)
    o_ref[...] = t1 - t2


def kernel(values, bin_values):
    r0, r1 = values.shape
    nbins = bin_values.shape[0]
    bins = bin_values
    u1i = 1.0 / (bins[1:] - bins[:-1])
    u1 = jnp.concatenate([jnp.zeros((1,), jnp.float32), u1i])
    au1 = jnp.concatenate([jnp.full((1,), -1.0, jnp.float32), bins[:-1] * u1i])
    nxt = jnp.concatenate([bins[1:], bins[-1:]])
    d2 = nxt - bins
    u2 = jnp.where(d2 > 0, 1.0 / jnp.maximum(d2, 1e-30), 0.0)
    au2 = bins * u2

    B = 32
    assert r0 % B == 0
    grid = r0 // B
    v3 = values.reshape(r0, r1, 1)

    def c3(x):
        return x.reshape(1, 1, nbins)

    cspec = pl.BlockSpec((1, 1, nbins), lambda i: (0, 0, 0))
    out = pl.pallas_call(
        _twohot_body,
        grid=(grid,),
        in_specs=[
            pl.BlockSpec((B, r1, 1), lambda i: (i, 0, 0)),
            cspec, cspec, cspec, cspec, cspec,
        ],
        out_specs=pl.BlockSpec((B, r1, nbins), lambda i: (i, 0, 0)),
        out_shape=jax.ShapeDtypeStruct((r0, r1, nbins), jnp.float32),
        compiler_params=pltpu.CompilerParams(
            dimension_semantics=("arbitrary",),
        ),
    )(v3, c3(bins), c3(u1), c3(au1), c3(u2), c3(au2))
    return out


# speedup vs baseline: 1.1293x; 1.1293x over previous
import functools

import jax
import jax.numpy as jnp
from jax.experimental import pallas as pl
from jax.experimental.pallas import tpu as pltpu

_NBUF = 4


def _twohot_body(v_ref, b_ref, u1_ref, au1_ref, u2_ref, au2_ref,
                 o_ref, buf_ref, sems, *, nblocks, blk):
    i = pl.program_id(0)
    slot = jax.lax.rem(i, _NBUF)

    @pl.when(i >= _NBUF)
    def _wait_prior():
        pltpu.make_async_copy(
            buf_ref.at[slot], o_ref.at[pl.ds(0, blk)], sems.at[slot]).wait()

    v = jnp.maximum(v_ref[...], b_ref[0, 0, 0])
    t1 = jnp.clip(v * u1_ref[...] - au1_ref[...], 0.0, 1.0)
    t2 = jnp.clip(v * u2_ref[...] - au2_ref[...], 0.0, 1.0)
    buf_ref[slot] = t1 - t2

    pltpu.make_async_copy(
        buf_ref.at[slot], o_ref.at[pl.ds(i * blk, blk)], sems.at[slot]).start()

    @pl.when(i == nblocks - 1)
    def _drain():
        for k in range(_NBUF):
            pltpu.make_async_copy(
                buf_ref.at[k], o_ref.at[pl.ds(0, blk)], sems.at[k]).wait()


def kernel(values, bin_values):
    r0, r1 = values.shape
    nbins = bin_values.shape[0]
    bins = bin_values
    u1i = 1.0 / (bins[1:] - bins[:-1])
    u1 = jnp.concatenate([jnp.zeros((1,), jnp.float32), u1i])
    au1 = jnp.concatenate([jnp.full((1,), -1.0, jnp.float32), bins[:-1] * u1i])
    nxt = jnp.concatenate([bins[1:], bins[-1:]])
    d2 = nxt - bins
    u2 = jnp.where(d2 > 0, 1.0 / jnp.maximum(d2, 1e-30), 0.0)
    au2 = bins * u2

    blk = 64
    assert r0 % blk == 0
    grid = r0 // blk
    v3 = values.reshape(r0, r1, 1)

    def c3(x):
        return x.reshape(1, 1, nbins)

    cspec = pl.BlockSpec((1, 1, nbins), lambda i: (0, 0, 0))
    out = pl.pallas_call(
        functools.partial(_twohot_body, nblocks=grid, blk=blk),
        grid=(grid,),
        in_specs=[
            pl.BlockSpec((blk, r1, 1), lambda i: (i, 0, 0)),
            cspec, cspec, cspec, cspec, cspec,
        ],
        out_specs=pl.BlockSpec(memory_space=pltpu.MemorySpace.HBM),
        out_shape=jax.ShapeDtypeStruct((r0, r1, nbins), jnp.float32),
        scratch_shapes=[
            pltpu.VMEM((_NBUF, blk, r1, nbins), jnp.float32),
            pltpu.SemaphoreType.DMA((_NBUF,)),
        ],
        compiler_params=pltpu.CompilerParams(
            dimension_semantics=("arbitrary",),
        ),
    )(v3, c3(bins), c3(u1), c3(au1), c3(u2), c3(au2))
    return out
